# baseline (speedup 1.0000x reference)
import functools

import jax
import jax.numpy as jnp
from jax import lax
from jax.experimental import pallas as pl
from jax.experimental.pallas import tpu as pltpu
from jax.experimental.pallas import tpu_sc as plsc

VOCAB = 150528
D_MODEL = 1024
BATCH = 4
SEQ = 2048
MASK_TOKEN = 150001

_INFO = plsc.get_sparse_core_info()
_NW = _INFO.num_cores * _INFO.num_subcores
_ROWS = BATCH * SEQ
_RPW = _ROWS // _NW
_CHUNK = 32
_NCH = _RPW // _CHUNK

_mesh = plsc.VectorSubcoreMesh(core_axis_name="c", subcore_axis_name="s")


_NPIECE = 1
_PROWS = _ROWS // _NPIECE
_PNCH = _PROWS // _NW // _CHUNK


def _make_sc_gather(nch):
    @functools.partial(
        pl.kernel,
        mesh=_mesh,
        out_type=jax.ShapeDtypeStruct((_ROWS, D_MODEL), jnp.float32),
        scratch_types=[
            pltpu.VMEM((nch, _CHUNK), jnp.int32),
            pltpu.VMEM((_CHUNK, D_MODEL), jnp.float32),
            pltpu.VMEM((_CHUNK, D_MODEL), jnp.float32),
            pltpu.SemaphoreType.DMA,
            pltpu.SemaphoreType.DMA,
            pltpu.SemaphoreType.DMA,
            pltpu.SemaphoreType.DMA,
        ],
    )
    def _sc_gather(idx_hbm, w_hbm, out_hbm, idx_v, buf0, buf1, si0, si1, so0,
                   so1):
        wid = lax.axis_index("s") * _INFO.num_cores + lax.axis_index("c")
        pltpu.sync_copy(idx_hbm.at[wid], idx_v)
        base = wid * _RPW
        bufs = (buf0, buf1)
        sin = (si0, si1)
        sout = (so0, so1)
        cin = [None] * nch
        cout = [None] * nch
        cin[0] = pltpu.async_copy(w_hbm.at[idx_v.at[0]], buf0, si0)
        if nch > 1:
            cin[1] = pltpu.async_copy(w_hbm.at[idx_v.at[1]], buf1, si1)
        for c in range(nch):
            b = c % 2
            cin[c].wait()
            cout[c] = pltpu.async_copy(
                bufs[b], out_hbm.at[pl.ds(base + c * _CHUNK, _CHUNK)],
                sout[b])
            nxt = c + 2
            if nxt < nch:
                cout[c].wait()
                cin[nxt] = pltpu.async_copy(w_hbm.at[idx_v.at[nxt]], bufs[b],
                                            sin[b])
            else:
                cout[c].wait()

    return _sc_gather


_sc_gather_piece = _make_sc_gather(_PNCH)


_BS = 1024


def _mask_body(ids_ref, amask_ref, pos_ref):
    sb = pl.program_id(1)
    ids = ids_ref[0, 0, :]
    col1 = lax.broadcasted_iota(jnp.int32, (1, SEQ), 1)
    mp = jnp.min(jnp.where(ids[None, :] == MASK_TOKEN, col1, SEQ))
    th = jnp.maximum(sb * _BS + lax.broadcasted_iota(jnp.int32, (_BS, 1), 0),
                     mp)
    cols = lax.broadcasted_iota(jnp.int32, (_BS, SEQ), 1)
    amask_ref[0, 0] = (cols > th).astype(jnp.int8)
    pos_ref[0] = jnp.minimum(col1, mp)


def _tc_mask(input_ids):
    amask, pos = pl.pallas_call(
        _mask_body,
        grid=(BATCH, SEQ // _BS),
        in_specs=[pl.BlockSpec((1, 1, SEQ), lambda b, sb: (b, 0, 0))],
        out_specs=[
            pl.BlockSpec((1, 1, _BS, SEQ), lambda b, sb: (b, 0, sb, 0)),
            pl.BlockSpec((1, 1, SEQ), lambda b, sb: (b, 0, 0)),
        ],
        out_shape=[
            jax.ShapeDtypeStruct((BATCH, 1, SEQ, SEQ), jnp.int8),
            jax.ShapeDtypeStruct((BATCH, 1, SEQ), jnp.int32),
        ],
    )(input_ids.reshape(BATCH, 1, SEQ))
    return amask.astype(jnp.bool_), pos.reshape(BATCH, SEQ)


_MBS = 256


def _merge_body(bsd_ref, out_ref):
    out_ref[...] = jnp.transpose(bsd_ref[...], (1, 0, 2))


def _tc_merge(bsd):
    return pl.pallas_call(
        _merge_body,
        grid=(SEQ // _MBS,),
        in_specs=[pl.BlockSpec((BATCH, _MBS, D_MODEL), lambda m: (0, m, 0))],
        out_specs=pl.BlockSpec((_MBS, BATCH, D_MODEL), lambda m: (m, 0, 0)),
        out_shape=jax.ShapeDtypeStruct((SEQ, BATCH, D_MODEL), jnp.float32),
    )(bsd)


def kernel(input_ids, labels, weight):
    attention_mask, position_ids = _tc_mask(input_ids)
    idx = input_ids.reshape(_NW, _PNCH, _CHUNK)
    flat = _sc_gather_piece(idx, weight)
    hidden_states = _tc_merge(flat.reshape(BATCH, SEQ, D_MODEL))
    return (hidden_states, position_ids, attention_mask, labels)


# speedup vs baseline: 1.0084x; 1.0084x over previous
import functools

import jax
import jax.numpy as jnp
from jax import lax
from jax.experimental import pallas as pl
from jax.experimental.pallas import tpu as pltpu
from jax.experimental.pallas import tpu_sc as plsc

VOCAB = 150528
D_MODEL = 1024
BATCH = 4
SEQ = 2048
MASK_TOKEN = 150001

_INFO = plsc.get_sparse_core_info()
_NW = _INFO.num_cores * _INFO.num_subcores
_ROWS = BATCH * SEQ
_RPW = _ROWS // _NW
_CHUNK = 32
_NCH = _RPW // _CHUNK

_mesh = plsc.VectorSubcoreMesh(core_axis_name="c", subcore_axis_name="s")


_NPIECE = 1
_PROWS = _ROWS // _NPIECE
_PNCH = _PROWS // _NW // _CHUNK


def _make_sc_gather(nch):
    @functools.partial(
        pl.kernel,
        mesh=_mesh,
        out_type=jax.ShapeDtypeStruct((_ROWS, D_MODEL), jnp.float32),
        scratch_types=[
            pltpu.VMEM((nch, _CHUNK), jnp.int32),
            pltpu.VMEM((_CHUNK, D_MODEL), jnp.float32),
            pltpu.VMEM((_CHUNK, D_MODEL), jnp.float32),
            pltpu.SemaphoreType.DMA,
            pltpu.SemaphoreType.DMA,
            pltpu.SemaphoreType.DMA,
            pltpu.SemaphoreType.DMA,
        ],
    )
    def _sc_gather(idx_hbm, w_hbm, out_hbm, idx_v, buf0, buf1, si0, si1, so0,
                   so1):
        wid = lax.axis_index("s") * _INFO.num_cores + lax.axis_index("c")
        pltpu.sync_copy(idx_hbm.at[wid], idx_v)
        base = wid * _RPW
        bufs = (buf0, buf1)
        sin = (si0, si1)
        sout = (so0, so1)
        cin = [None] * nch
        cout = [None] * nch
        cin[0] = pltpu.async_copy(w_hbm.at[idx_v.at[0]], buf0, si0)
        if nch > 1:
            cin[1] = pltpu.async_copy(w_hbm.at[idx_v.at[1]], buf1, si1)
        for c in range(nch):
            b = c % 2
            cin[c].wait()
            cout[c] = pltpu.async_copy(
                bufs[b], out_hbm.at[pl.ds(base + c * _CHUNK, _CHUNK)],
                sout[b])
            nxt = c + 2
            if nxt < nch:
                cout[c].wait()
                cin[nxt] = pltpu.async_copy(w_hbm.at[idx_v.at[nxt]], bufs[b],
                                            sin[b])
            else:
                cout[c].wait()

    return _sc_gather


_sc_gather_piece = _make_sc_gather(_PNCH)


_BS = 1024


def _mask_body(ids_ref, amask_ref, pos_ref):
    sb = pl.program_id(1)
    ids = ids_ref[0, 0, :]
    col1 = lax.broadcasted_iota(jnp.int32, (1, SEQ), 1)
    mp = jnp.min(jnp.where(ids[None, :] == MASK_TOKEN, col1, SEQ))
    th = jnp.maximum(sb * _BS + lax.broadcasted_iota(jnp.int32, (_BS, 1), 0),
                     mp)
    cols = lax.broadcasted_iota(jnp.int32, (_BS, SEQ), 1)
    amask_ref[0, 0] = (cols > th).astype(jnp.int8)
    pos_ref[0] = jnp.minimum(col1, mp)


def _tc_mask(input_ids):
    amask, pos = pl.pallas_call(
        _mask_body,
        grid=(BATCH, SEQ // _BS),
        in_specs=[pl.BlockSpec((1, 1, SEQ), lambda b, sb: (b, 0, 0))],
        out_specs=[
            pl.BlockSpec((1, 1, _BS, SEQ), lambda b, sb: (b, 0, sb, 0)),
            pl.BlockSpec((1, 1, SEQ), lambda b, sb: (b, 0, 0)),
        ],
        out_shape=[
            jax.ShapeDtypeStruct((BATCH, 1, SEQ, SEQ), jnp.int8),
            jax.ShapeDtypeStruct((BATCH, 1, SEQ), jnp.int32),
        ],
    )(input_ids.reshape(BATCH, 1, SEQ))
    return amask.astype(jnp.bool_), pos.reshape(BATCH, SEQ)


_MBS = 512


def _merge_body(bsd_ref, out_ref):
    out_ref[...] = jnp.transpose(bsd_ref[...], (1, 0, 2))


def _tc_merge(bsd):
    return pl.pallas_call(
        _merge_body,
        grid=(SEQ // _MBS,),
        in_specs=[pl.BlockSpec((BATCH, _MBS, D_MODEL), lambda m: (0, m, 0))],
        out_specs=pl.BlockSpec((_MBS, BATCH, D_MODEL), lambda m: (m, 0, 0)),
        out_shape=jax.ShapeDtypeStruct((SEQ, BATCH, D_MODEL), jnp.float32),
    )(bsd)


def kernel(input_ids, labels, weight):
    attention_mask, position_ids = _tc_mask(input_ids)
    idx = input_ids.reshape(_NW, _PNCH, _CHUNK)
    flat = _sc_gather_piece(idx, weight)
    hidden_states = _tc_merge(flat.reshape(BATCH, SEQ, D_MODEL))
    return (hidden_states, position_ids, attention_mask, labels)
